# Initial kernel scaffold; baseline (speedup 1.0000x reference)
import jax
import jax.numpy as jnp
from jax.experimental import pallas as pl


def kernel(params, clauses, edge_lit, edge_clause):
    raise NotImplementedError("write your pallas kernel here")


# speedup vs baseline: 2.8580x; 2.8580x over previous
import functools

import jax
import jax.numpy as jnp
from jax import lax
from jax.experimental import pallas as pl
from jax.experimental.pallas import tpu as pltpu
from jax.experimental.pallas import tpu_sc as plsc

NV = 5000
NL = 2 * NV
C = 100000
K = 3
F = 128
QD = 64
ROUNDS = 4

NC, NS = 2, 16
NW = NC * NS
CPW = 3200
C_PAD = NW * CPW
E_PAD = 3 * C_PAD
CH = 128
NCH = CPW // CH
EPW = 3 * CPW
ECH = 128
NECH = EPW // ECH
TPT = 624
ZB = 208
DUMP = NL
EPS_DENOM = 1e-9
EPS_LOG = 1e-8

@functools.cache
def _mesh():
    return plsc.VectorSubcoreMesh(core_axis_name="c", subcore_axis_name="s",
                                  num_cores=NC, num_subcores=NS)


def _wid():
    return lax.axis_index("s") * NC + lax.axis_index("c")


def _zero_rows(zbuf, nrow, d):
    z = jnp.zeros((16,), jnp.float32)

    def body(r, _):
        for j in range(d // 16):
            zbuf[r, pl.ds(16 * j, 16)] = z
        return 0

    lax.fori_loop(0, nrow, body, 0)


def _zero_shared(shared, zbuf, d):
    _zero_rows(zbuf, ZB, d)
    sid = lax.axis_index("s")
    for p in range(3):
        pltpu.sync_copy(zbuf, shared.at[pl.ds(sid * TPT + p * ZB, ZB)])

    @pl.when(sid == 0)
    def _():
        pltpu.sync_copy(zbuf.at[pl.ds(0, 16)], shared.at[pl.ds(NS * TPT, 16)])


def _flush_shared(shared, out_hbm, d):
    sid = lax.axis_index("s")
    cid = lax.axis_index("c")
    pltpu.sync_copy(shared.at[pl.ds(sid * TPT, TPT)],
                    out_hbm.at[cid, pl.ds(sid * TPT, TPT)])

    @pl.when(sid == 0)
    def _():
        pltpu.sync_copy(shared.at[pl.ds(NS * TPT, 16)],
                        out_hbm.at[cid, pl.ds(NS * TPT, 16)])


@functools.cache
def _build_sc_scatter64():
    d = QD

    @functools.partial(
        pl.kernel,
        out_type=jax.ShapeDtypeStruct((2, NL, d), jnp.float32),
        mesh=_mesh(),
        compiler_params=pltpu.CompilerParams(use_tc_tiling_on_sc=False),
        scratch_types=[
            pltpu.VMEM((CH,), jnp.int32),
            pltpu.VMEM((CH, d), jnp.float32),
            pltpu.VMEM((ZB, d), jnp.float32),
            pltpu.VMEM_SHARED((NL + 8, d), jnp.float32),
            pltpu.SemaphoreType.DMA,
        ],
    )
    def sc_s64(s0h, s1h, s2h, i0h, i1h, i2h, dep, out, iv, sb, zbuf, sh, sem):
        del dep
        _zero_shared(sh, zbuf, d)
        plsc.subcore_barrier()
        wid = _wid()

        def chunk(ci, _):
            base = wid * CPW + ci * CH
            for sh_src, ih in ((s0h, i0h), (s1h, i1h), (s2h, i2h)):
                pltpu.sync_copy(ih.at[pl.ds(base, CH)], iv)
                pltpu.sync_copy(sh_src.at[pl.ds(base, CH)], sb)
                pltpu.sync_copy(sb, sh.at[iv], add=True)
            return 0

        lax.fori_loop(0, NCH, chunk, 0)
        plsc.subcore_barrier()
        _flush_shared(sh, out, d)

    return sc_s64


@functools.cache
def _build_sc_wscatter64():
    d = QD

    @functools.partial(
        pl.kernel,
        out_type=jax.ShapeDtypeStruct((2, NL, d), jnp.float32),
        mesh=_mesh(),
        compiler_params=pltpu.CompilerParams(use_tc_tiling_on_sc=False),
        scratch_types=[
            pltpu.VMEM((CH,), jnp.int32),
            pltpu.VMEM((CH, d), jnp.float32),
            pltpu.VMEM((CH, d), jnp.float32),
            pltpu.VMEM((CH, d), jnp.float32),
            pltpu.VMEM((ZB, d), jnp.float32),
            pltpu.VMEM_SHARED((NL + 8, d), jnp.float32),
            pltpu.SemaphoreType.DMA,
        ],
    )
    def sc_ws64(vmh, ex0, ex1, ex2, i0h, i1h, i2h, dep, out,
                iv, vmb, exb, wb, zbuf, sh, sem):
        del dep
        _zero_shared(sh, zbuf, d)
        plsc.subcore_barrier()
        wid = _wid()

        def chunk(ci, _):
            base = wid * CPW + ci * CH
            pltpu.sync_copy(vmh.at[pl.ds(base, CH)], vmb)
            for exh, ih in ((ex0, i0h), (ex1, i1h), (ex2, i2h)):
                pltpu.sync_copy(ih.at[pl.ds(base, CH)], iv)
                pltpu.sync_copy(exh.at[pl.ds(base, CH)], exb)

                def row(r, _):
                    for j in range(d // 16):
                        sl = pl.ds(16 * j, 16)
                        wb[r, sl] = vmb[r, sl] * exb[r, sl]
                    return 0

                lax.fori_loop(0, CH, row, 0)
                pltpu.sync_copy(wb, sh.at[iv], add=True)
            return 0

        lax.fori_loop(0, NCH, chunk, 0)
        plsc.subcore_barrier()
        _flush_shared(sh, out, d)

    return sc_ws64


@functools.cache
def _build_sc_gather_sum3(d):
    @functools.partial(
        pl.kernel,
        out_type=jax.ShapeDtypeStruct((C_PAD, d), jnp.float32),
        mesh=_mesh(),
        compiler_params=pltpu.CompilerParams(use_tc_tiling_on_sc=False),
        scratch_types=[
            pltpu.VMEM((CH,), jnp.int32),
            pltpu.VMEM((CH,), jnp.int32),
            pltpu.VMEM((CH,), jnp.int32),
            pltpu.VMEM((CH, d), jnp.float32),
            pltpu.VMEM((CH, d), jnp.float32),
            pltpu.VMEM((CH, d), jnp.float32),
            pltpu.VMEM((CH, d), jnp.float32),
            pltpu.SemaphoreType.DMA,
        ],
    )
    def sc_gs3(table, i0h, i1h, i2h, out, i0, i1, i2, r0, r1, r2, acc, sem):
        wid = _wid()

        def chunk(ci, _):
            base = wid * CPW + ci * CH
            pltpu.sync_copy(i0h.at[pl.ds(base, CH)], i0)
            pltpu.sync_copy(i1h.at[pl.ds(base, CH)], i1)
            pltpu.sync_copy(i2h.at[pl.ds(base, CH)], i2)
            pltpu.async_copy(table.at[i0], r0, sem).wait()
            pltpu.async_copy(table.at[i1], r1, sem).wait()
            pltpu.async_copy(table.at[i2], r2, sem).wait()

            def row(r, _):
                for j in range(d // 16):
                    sl = pl.ds(16 * j, 16)
                    acc[r, sl] = r0[r, sl] + r1[r, sl] + r2[r, sl]
                return 0

            lax.fori_loop(0, CH, row, 0)
            pltpu.sync_copy(acc, out.at[pl.ds(base, CH)])
            return 0

        lax.fori_loop(0, NCH, chunk, 0)

    return sc_gs3


@functools.cache
def _build_sc_gather_flat(d):
    @functools.partial(
        pl.kernel,
        out_type=jax.ShapeDtypeStruct((E_PAD, d), jnp.float32),
        mesh=_mesh(),
        compiler_params=pltpu.CompilerParams(use_tc_tiling_on_sc=False),
        scratch_types=[
            pltpu.VMEM((ECH,), jnp.int32),
            pltpu.VMEM((ECH, d), jnp.float32),
            pltpu.SemaphoreType.DMA,
        ],
    )
    def sc_gf(table, idxg, dep, out, iv, rows, sem):
        del dep
        wid = _wid()

        def chunk(ci, _):
            base = wid * EPW + ci * ECH
            pltpu.sync_copy(idxg.at[pl.ds(base, ECH)], iv)
            pltpu.async_copy(table.at[iv], rows, sem).wait()
            pltpu.sync_copy(rows, out.at[pl.ds(base, ECH)])
            return 0

        lax.fori_loop(0, NECH, chunk, 0)

    return sc_gf


def _dot(a, b):
    return jnp.dot(a, b, preferred_element_type=jnp.float32)


def _ln(x):
    m = jnp.mean(x, axis=-1, keepdims=True)
    v = jnp.var(x, axis=-1, keepdims=True)
    return (x - m) * lax.rsqrt(v + 1e-6)


def _full(shape):
    return pl.BlockSpec(shape, lambda i: tuple(0 for _ in shape))


BN = 1000
BV = 1000
BC = 2048


@functools.cache
def _build_t_query():
    def body(lo_a, no_a, lo_b, no_b, w0, b0, w1, b1, w2, b2, out):
        x = _dot(jnp.concatenate(
            [lo_a[...], no_a[...], lo_b[...], no_b[...]], axis=-1),
            w0[...]) + b0[...]
        h = jax.nn.relu(_ln(x))
        h = jax.nn.relu(_ln(_dot(h, w1[...]) + b1[...]))
        q = _dot(h, w2[...]) + b2[...]
        out[...] = jnp.stack([q, -q])

    nb = NV // BV
    return pl.pallas_call(
        body,
        grid=(nb,),
        in_specs=[
            pl.BlockSpec((BV, F), lambda i: (i, 0)),
            pl.BlockSpec((BV, 4), lambda i: (i, 0)),
            pl.BlockSpec((BV, F), lambda i: (i + nb, 0)),
            pl.BlockSpec((BV, 4), lambda i: (i + nb, 0)),
            _full((2 * F + 8, F)),
            _full((1, F)), _full((F, F)), _full((1, F)),
            _full((F, QD)), _full((1, QD)),
        ],
        out_specs=pl.BlockSpec((2, BV, QD), lambda i: (0, i, 0)),
        out_shape=jax.ShapeDtypeStruct((2, NV, QD), jnp.float32),
    )


@functools.cache
def _build_t_ll():
    def body(lbuf, out):
        out[...] = lbuf[0] + lbuf[1]

    return pl.pallas_call(
        body,
        grid=(NL // BN,),
        in_specs=[pl.BlockSpec((2, BN, QD), lambda i: (0, i, 0))],
        out_specs=pl.BlockSpec((BN, QD), lambda i: (i, 0)),
        out_shape=jax.ShapeDtypeStruct((NL, QD), jnp.float32),
    )


@functools.cache
def _build_t_lg():
    nb = NV // BV

    def body(ga, gb, out):
        qg = (ga[0] + ga[1]) - (gb[0] + gb[1])
        out[...] = jnp.stack([qg[:, :QD // 2], qg[:, QD // 2:]])

    return pl.pallas_call(
        body,
        grid=(nb,),
        in_specs=[
            pl.BlockSpec((2, BV, QD), lambda i: (0, i, 0)),
            pl.BlockSpec((2, BV, QD), lambda i: (0, i + nb, 0)),
        ],
        out_specs=pl.BlockSpec((2, BV, QD // 2), lambda i: (0, i, 0)),
        out_shape=jax.ShapeDtypeStruct((2, NV, QD // 2), jnp.float32),
    )


@functools.cache
def _build_t_mm(n, k, m, bn):
    def body(x, w, b, o):
        o[...] = _dot(x[...], w[...]) + b[...]

    return pl.pallas_call(
        body,
        grid=(n // bn,),
        in_specs=[pl.BlockSpec((bn, k), lambda i: (i, 0)),
                  _full((k, m)), _full((1, m))],
        out_specs=pl.BlockSpec((bn, m), lambda i: (i, 0)),
        out_shape=jax.ShapeDtypeStruct((n, m), jnp.float32),
    )


@functools.cache
def _build_t_litnorm():
    def body(h, ln_out):
        hh = h[...]
        m0 = jnp.mean(hh, axis=0, keepdims=True)
        v0 = jnp.mean((hh - m0) * (hh - m0), axis=0, keepdims=True)
        ln_out[...] = (hh - m0) * lax.rsqrt(v0 + 1e-6)

    return pl.pallas_call(
        body,
        grid=(1,),
        in_specs=[_full((NL, F))],
        out_specs=_full((NL, F)),
        out_shape=jax.ShapeDtypeStruct((NL, F), jnp.float32),
    )


def _lnorm(x, eps=1e-6):
    m = jnp.mean(x, axis=-1, keepdims=True)
    v = jnp.var(x, axis=-1, keepdims=True)
    return (x - m) * lax.rsqrt(v + eps)


def _mlp_pallas(x, Ws, bs, bn):
    n = x.shape[0]
    for W, b in zip(Ws[:-1], bs[:-1]):
        y = _build_t_mm(n, W.shape[0], W.shape[1], bn)(x, W, b.reshape(1, -1))
        x = jax.nn.relu(_lnorm(y))
    W, b = Ws[-1], bs[-1]
    return _build_t_mm(n, W.shape[0], W.shape[1], bn)(x, W, b.reshape(1, -1))


@functools.cache
def _build_t_closs():
    def body(vals, out_l, out_g0, out_g1, out_g2):
        v = vals[...].reshape(BC, K, QD)
        cl, vjp = jax.vjp(
            lambda x: jnp.exp(-jnp.sum(jax.nn.softplus(x), axis=1)), v)
        (gv,) = vjp(jnp.ones_like(cl))
        out_l[...] = cl
        out_g0[...] = gv[:, 0]
        out_g1[...] = gv[:, 1]
        out_g2[...] = gv[:, 2]

    return pl.pallas_call(
        body,
        grid=(C_PAD // BC,),
        in_specs=[pl.BlockSpec((K * BC, QD), lambda i: (i, 0))],
        out_specs=tuple(pl.BlockSpec((BC, QD), lambda i: (i, 0))
                        for _ in range(4)),
        out_shape=tuple(jax.ShapeDtypeStruct((C_PAD, QD), jnp.float32)
                        for _ in range(4)),
    )


@functools.cache
def _build_t_unit_mm(nrows, bn):
    def body(lo, g, l_, w, o1):
        o1[...] = _dot(jnp.concatenate([lo[...], g[...], l_[...]], axis=-1),
                       w[...])

    return pl.pallas_call(
        body,
        grid=(nrows // bn,),
        in_specs=[
            pl.BlockSpec((bn, F), lambda i: (i, 0)),
            pl.BlockSpec((bn, QD // 2), lambda i: (i, 0)),
            pl.BlockSpec((bn, QD), lambda i: (i, 0)),
            _full((F + QD // 2 + QD, F)),
        ],
        out_specs=pl.BlockSpec((bn, F), lambda i: (i, 0)),
        out_shape=jax.ShapeDtypeStruct((nrows, F), jnp.float32),
    )


@functools.cache
def _build_t_mv():
    def body(cf, cg, cl, wm, wv, o_mp, o_va, o_vb):
        cu = jnp.concatenate([cf[...], cg[...], cl[...]], axis=-1)
        o_mp[...] = _dot(cu, wm[...])
        vm = _dot(cu, wv[...])
        o_va[...] = vm[:, :QD]
        o_vb[...] = vm[:, QD:]

    return pl.pallas_call(
        body,
        grid=(C_PAD // BC,),
        in_specs=[
            pl.BlockSpec((BC, F), lambda i: (i, 0)),
            pl.BlockSpec((BC, QD // 2), lambda i: (i, 0)),
            pl.BlockSpec((BC, QD), lambda i: (i, 0)),
            _full((F + QD // 2 + QD, F)), _full((F + QD // 2 + QD, F)),
        ],
        out_specs=(pl.BlockSpec((BC, F), lambda i: (i, 0)),
                   pl.BlockSpec((BC, QD), lambda i: (i, 0)),
                   pl.BlockSpec((BC, QD), lambda i: (i, 0))),
        out_shape=(jax.ShapeDtypeStruct((C_PAD, F), jnp.float32),
                   jax.ShapeDtypeStruct((C_PAD, QD), jnp.float32),
                   jax.ShapeDtypeStruct((C_PAD, QD), jnp.float32)),
    )


@functools.cache
def _build_t_ex():
    def body(qpe, mp, av, o0, o1, o2):
        t = qpe[...].reshape(BC, K, F) + mp[...][:, None, :]
        s = jnp.sum(jnp.tanh(t) * av[...][0][None, None, :], axis=-1)
        ex = jnp.exp(s)
        for k, o in enumerate((o0, o1, o2)):
            o[...] = jnp.broadcast_to(ex[:, k:k + 1], (BC, QD))

    return pl.pallas_call(
        body,
        grid=(C_PAD // BC,),
        in_specs=[
            pl.BlockSpec((K * BC, F), lambda i: (i, 0)),
            pl.BlockSpec((BC, F), lambda i: (i, 0)),
            _full((1, F)),
        ],
        out_specs=tuple(pl.BlockSpec((BC, QD), lambda i: (i, 0))
                        for _ in range(K)),
        out_shape=tuple(jax.ShapeDtypeStruct((C_PAD, QD), jnp.float32)
                        for _ in range(K)),
    )


@functools.cache
def _build_t_lmlp():
    def body(acca, accb, accex, lo, g, l_, wa, b0, w1, b1, w2, b2, out):
        asum = jnp.concatenate([acca[0] + acca[1], accb[0] + accb[1]], axis=-1)
        esum = accex[0] + accex[1]
        nl = asum / jnp.maximum(esum[:, 0:1], 1e-30)
        x = _dot(jnp.concatenate([lo[...], g[...], l_[...], nl], axis=-1),
                 wa[...]) + b0[...]
        h = jax.nn.relu(_ln(x))
        h = jax.nn.relu(_ln(_dot(h, w1[...]) + b1[...]))
        out[...] = _dot(h, w2[...]) + b2[...]

    return pl.pallas_call(
        body,
        grid=(NL // BN,),
        in_specs=[
            pl.BlockSpec((2, BN, QD), lambda i: (0, i, 0)),
            pl.BlockSpec((2, BN, QD), lambda i: (0, i, 0)),
            pl.BlockSpec((2, BN, QD), lambda i: (0, i, 0)),
            pl.BlockSpec((BN, F), lambda i: (i, 0)),
            pl.BlockSpec((BN, QD // 2), lambda i: (i, 0)),
            pl.BlockSpec((BN, QD), lambda i: (i, 0)),
            _full((2 * F + QD // 2 + QD, F)),
            _full((1, F)), _full((F, F)), _full((1, F)), _full((F, F)),
            _full((1, F)),
        ],
        out_specs=pl.BlockSpec((BN, F), lambda i: (i, 0)),
        out_shape=jax.ShapeDtypeStruct((NL, F), jnp.float32),
    )


@functools.cache
def _build_t_norm_omlp():
    def body(h, w0a, b0, w1, b1, w2, b2, ln_out, logits_out, rep_out):
        hh = h[...]
        m0 = jnp.mean(hh, axis=0, keepdims=True)
        v0 = jnp.mean((hh - m0) * (hh - m0), axis=0, keepdims=True)
        ln = (hh - m0) * lax.rsqrt(v0 + 1e-6)
        ln_out[...] = ln
        x = _dot(jnp.concatenate([ln[:NV], ln[NV:]], axis=-1),
                 w0a[...]) + b0[...]
        hq = jax.nn.relu(_ln(x))
        hq = jax.nn.relu(_ln(_dot(hq, w1[...]) + b1[...]))
        logits = (_dot(hq, w2[...]) + b2[...]) * 0.25
        logits_out[...] = logits
        rep = jnp.broadcast_to(logits, (NV, 16))
        rep_out[...] = jnp.concatenate([rep, -rep], axis=0)

    return pl.pallas_call(
        body,
        grid=(1,),
        in_specs=[
            _full((NL, F)),
            _full((2 * F, F)), _full((1, F)),
            _full((F, F)), _full((1, F)), _full((F, 1)), _full((1, 1)),
        ],
        out_specs=(
            _full((NL, F)),
            _full((NV, 1)),
            _full((NL, 16)),
        ),
        out_shape=(
            jax.ShapeDtypeStruct((NL, F), jnp.float32),
            jax.ShapeDtypeStruct((NV, 1), jnp.float32),
            jax.ShapeDtypeStruct((NL, 16), jnp.float32),
        ),
    )


@functools.cache
def _build_t_loss():
    nb = C_PAD // BC

    def body(lv, out):
        i = pl.program_id(0)
        v = lv[...].reshape(BC, K, 16)[:, :, 0]
        cu = jnp.exp(-jnp.sum(jax.nn.softplus(v), axis=1))
        lc = -jnp.log(1.0 - cu + EPS_LOG)
        cglob = i * BC + lax.broadcasted_iota(jnp.int32, (BC,), 0)
        partial = jnp.sum(jnp.where(cglob < C, lc, 0.0))
        prev = jnp.where(i == 0, jnp.zeros((1, 1), jnp.float32), out[...])
        out[...] = prev + partial

    return pl.pallas_call(
        body,
        grid=(nb,),
        in_specs=[pl.BlockSpec((K * BC, 16), lambda i: (i, 0))],
        out_specs=pl.BlockSpec((1, 1), lambda i: (0, 0)),
        out_shape=jax.ShapeDtypeStruct((1, 1), jnp.float32),
    )


def _zero_state():
    onehot = jax.nn.one_hot(jnp.zeros([NL], dtype=jnp.int32), F)
    onehot = onehot - 1.0 / F
    return onehot * jnp.sqrt(jnp.float32(F)) * 0.25


def _pad_rows(x, n=8):
    return jnp.pad(x, ((0, n), (0, 0)))


def kernel(params, clauses, edge_lit, edge_clause):
    p = params
    idx3 = jnp.concatenate(
        [clauses.T, jnp.full((K, C_PAD - C), DUMP, jnp.int32)], axis=1)
    idxg = idx3.T.reshape(E_PAD)
    i0a, i1a, i2a = idx3[0], idx3[1], idx3[2]

    qW0, qW1, qW2 = p['q_Ws']
    qb0, qb1, qb2 = [b.reshape(1, -1) for b in p['q_bs']]
    AWq, AWm, AWv = p['AWq'], p['AWm'], p['AWv']
    av2 = p['Av'].reshape(1, F)
    lW0, lW1, lW2 = p['l_Ws']
    lb0, lb1, lb2 = [b.reshape(1, -1) for b in p['l_bs']]
    oW0, oW1, oW2 = p['o_Ws']
    ob0, ob1, ob2 = [b.reshape(1, -1) for b in p['o_bs']]

    sc_s64 = _build_sc_scatter64()
    sc_ws64 = _build_sc_wscatter64()
    sc_gs3_f = _build_sc_gather_sum3(F)
    sc_gs3_g = _build_sc_gather_sum3(QD // 2)
    sc_gf_qp = _build_sc_gather_flat(F)
    sc_gf_v = _build_sc_gather_flat(QD)
    sc_gf_lg = _build_sc_gather_flat(16)
    t_closs = _build_t_closs()
    t_ll = _build_t_ll()
    t_lg = _build_t_lg()
    t_qp = _build_t_unit_mm(NL, BN)
    t_mv = _build_t_mv()
    t_ex = _build_t_ex()
    t_litnorm = _build_t_litnorm()
    t_loss = _build_t_loss()

    l_output = _zero_state()
    dep8 = jnp.zeros((8, 8), jnp.float32)
    lv_dep = dep8
    logits = jnp.zeros((NV, 1), jnp.float32)
    step_losses = []
    nkey = jax.random.key(7)
    for _ in range(ROUNDS):
        nkey, sk = jax.random.split(nkey)
        noise = jax.random.normal(sk, (NL, 4), dtype=jnp.float32)

        lits = jnp.concatenate([l_output, noise], axis=-1)
        variables = jnp.concatenate([lits[:NV], lits[NV:]], axis=1)
        query = _mlp_pallas(variables, [qW0, qW1, qW2],
                            [qb0[0], qb1[0], qb2[0]], BV)
        lits_q = jnp.concatenate([query, -query], axis=0)

        vals = sc_gf_v(_pad_rows(lits_q), idxg, dep8)
        closs, cg0, cg1, cg2 = t_closs(vals)
        dep0 = jnp.zeros((2, 8, QD), jnp.float32)
        lbuf = sc_s64(closs, closs, closs, i0a, i1a, i2a, dep0)
        gbuf = sc_s64(cg0, cg1, cg2, i0a, i1a, i2a, lbuf[:, :8, :])
        ll = t_ll(lbuf)
        lg = t_lg(gbuf, gbuf).reshape(NL, QD // 2)

        cfull = sc_gs3_f(_pad_rows(l_output), i0a, i1a, i2a)
        cgrad = sc_gs3_g(_pad_rows(lg), i0a, i1a, i2a)

        qp = t_qp(l_output, lg, ll, AWq)
        mp, vma, vmb = t_mv(cfull, cgrad, closs, AWm, AWv)

        qpe = sc_gf_qp(_pad_rows(qp), idxg, dep8)
        ex0, ex1, ex2 = t_ex(qpe, mp, av2)
        acca = sc_ws64(vma, ex0, ex1, ex2, i0a, i1a, i2a, dep0)
        accb = sc_ws64(vmb, ex0, ex1, ex2, i0a, i1a, i2a, acca[:, :8, :])
        accex = sc_s64(ex0, ex1, ex2, i0a, i1a, i2a, gbuf[:, :8, :])

        asum = jnp.concatenate([acca[0] + acca[1], accb[0] + accb[1]], -1)
        esum = accex[0] + accex[1]
        nl = asum / jnp.maximum(esum[:, 0:1], 1e-30)
        lmlp_in = jnp.concatenate([l_output, lg, ll, nl], axis=-1)
        h = _mlp_pallas(lmlp_in, [lW0, lW1, lW2],
                        [lb0[0], lb1[0], lb2[0]], BN)
        l_output = t_litnorm(h)
        ovars = jnp.concatenate([l_output[:NV], l_output[NV:]], axis=1)
        logits = _mlp_pallas(ovars, [oW0, oW1, oW2],
                             [ob0[0], ob1[0], ob2[0]], BV) * 0.25
        rep = jnp.broadcast_to(logits, (NV, 16))
        llog_rep = jnp.concatenate([rep, -rep], axis=0)

        l_output = l_output * 0.2 + l_output * 0.8
        lv = sc_gf_lg(_pad_rows(llog_rep), idxg, lv_dep)
        lv_dep = lv[:8, :8]
        loss = t_loss(lv)[0, 0]
        step_losses.append(loss)

    return logits, jnp.mean(jnp.stack(step_losses))
